# Initial kernel scaffold; baseline (speedup 1.0000x reference)
import jax
import jax.numpy as jnp
from jax.experimental import pallas as pl


def kernel(graph, features, W1, b1, W2, b2, W3, b3):
    raise NotImplementedError("write your pallas kernel here")


# speedup vs baseline: 6.0475x; 6.0475x over previous
import functools

import jax
import jax.numpy as jnp
from jax import lax
from jax.experimental import pallas as pl
from jax.experimental.pallas import tpu as pltpu
from jax.experimental.pallas import tpu_sc as plsc

_B = 4
_P = 2048
_K = 20
_N = _B * _P
_E = 131072
_NW = 32
_NEG = -3.0e38


def _dist_topk_body(rows_ref, full_ref, y_ref, yblk_ref, idx_ref):
    b = pl.program_id(0)
    rows = rows_ref[0]
    full = full_ref[0]
    br = rows.shape[0]
    gram = lax.dot_general(rows, full, (((1,), (1,)), ((), ())),
                           preferred_element_type=jnp.float32)
    y = y_ref[0]
    sq_f = jnp.sum(y * y, axis=0, keepdims=True)
    yb = yblk_ref[0]
    sq_r = jnp.transpose(jnp.sum(yb * yb, axis=0, keepdims=True), (1, 0))
    pd = 2.0 * gram - sq_r - sq_f
    cols = lax.broadcasted_iota(jnp.int32, (br, _P), 1)
    picks = []
    for _ in range(_K):
        m = jnp.max(pd, axis=1, keepdims=True)
        cand = jnp.where(pd >= m, cols, _P)
        a = jnp.min(cand, axis=1, keepdims=True)
        picks.append(a)
        pd = jnp.where(cols == a, _NEG, pd)
    idx_ref[0] = jnp.concatenate(picks, axis=1) + b * _P


def _dist_topk(xt, y3):
    c = xt.shape[1]
    br = 256
    x3 = xt.reshape(_B, _P, c)
    out = pl.pallas_call(
        _dist_topk_body,
        grid=(_B, _P // br),
        in_specs=[
            pl.BlockSpec((1, br, c), lambda b, r: (b, r, 0)),
            pl.BlockSpec((1, _P, c), lambda b, r: (b, 0, 0)),
            pl.BlockSpec((1, c, _P), lambda b, r: (b, 0, 0)),
            pl.BlockSpec((1, c, br), lambda b, r: (b, 0, r)),
        ],
        out_specs=pl.BlockSpec((1, br, _K), lambda b, r: (b, r, 0)),
        out_shape=jax.ShapeDtypeStruct((_B, _P, _K), jnp.int32),
    )(x3, x3, y3, y3)
    return out.reshape(_N, _K)


def _feat_mm_body(mx_ref, xt_ref, w_ref, deg_ref, out_ref):
    c = xt_ref.shape[1]
    d = mx_ref[...] - xt_ref[...]
    h = lax.dot_general(d, w_ref[:c], (((1,), (0,)), ((), ())),
                        preferred_element_type=jnp.float32)
    h += lax.dot_general(xt_ref[...], w_ref[c:], (((1,), (0,)), ((), ())),
                         preferred_element_type=jnp.float32)
    norm = lax.rsqrt(jnp.maximum(deg_ref[...], 1.0))
    out_ref[...] = h * norm


def _feat_mm(mx, xt, w, deg):
    c = xt.shape[1]
    h = w.shape[1]
    bn = 512
    return pl.pallas_call(
        _feat_mm_body,
        grid=(_N // bn,),
        in_specs=[
            pl.BlockSpec((bn, c), lambda i: (i, 0)),
            pl.BlockSpec((bn, c), lambda i: (i, 0)),
            pl.BlockSpec((2 * c, h), lambda i: (0, 0)),
            pl.BlockSpec((bn, 1), lambda i: (i, 0)),
        ],
        out_specs=pl.BlockSpec((bn, h), lambda i: (i, 0)),
        out_shape=jax.ShapeDtypeStruct((_N, h), jnp.float32),
    )(mx, xt, w, deg)


def _epi_body(t0_ref, t1_ref, deg_ref, b_ref, out_ref, *, relu):
    norm = lax.rsqrt(jnp.maximum(deg_ref[...], 1.0))
    o = (t0_ref[...] + t1_ref[...]) * norm + b_ref[...]
    if relu:
        o = jnp.maximum(o, 0.0)
    out_ref[...] = o


def _epilogue(t0, t1, deg, bias, relu):
    h = t0.shape[1]
    bn = 512
    return pl.pallas_call(
        functools.partial(_epi_body, relu=relu),
        grid=(_N // bn,),
        in_specs=[
            pl.BlockSpec((bn, h), lambda i: (i, 0)),
            pl.BlockSpec((bn, h), lambda i: (i, 0)),
            pl.BlockSpec((bn, 1), lambda i: (i, 0)),
            pl.BlockSpec((1, h), lambda i: (0, 0)),
        ],
        out_specs=pl.BlockSpec((bn, h), lambda i: (i, 0)),
        out_shape=jax.ShapeDtypeStruct((_N, h), jnp.float32),
    )(t0, t1, deg, bias)


def _gather_max(xt, gidx):
    c = xt.shape[1]
    chunk = 16
    gsub = 80
    rows_per_chunk = chunk * _K
    pts_per_w = _N // _NW
    nchunks = pts_per_w // chunk
    mesh = plsc.VectorSubcoreMesh(core_axis_name="c", subcore_axis_name="s")

    @functools.partial(
        pl.kernel,
        mesh=mesh,
        out_type=jax.ShapeDtypeStruct((_N, c), jnp.float32),
        scratch_types=[
            pltpu.VMEM((rows_per_chunk,), jnp.int32),
            pltpu.VMEM((rows_per_chunk, c), jnp.float32),
            pltpu.VMEM((chunk, c), jnp.float32),
            pltpu.SemaphoreType.DMA,
        ],
    )
    def k(idx_hbm, xt_hbm, out_hbm, idx_v, rows_v, out_v, sem):
        wid = lax.axis_index("s") * 2 + lax.axis_index("c")
        base = wid * pts_per_w

        def chunk_body(ci, _):
            p0 = base + ci * chunk
            pltpu.sync_copy(idx_hbm.at[pl.ds(p0 * _K, rows_per_chunk)], idx_v)
            for g in range(rows_per_chunk // gsub):
                pltpu.async_copy(
                    xt_hbm.at[idx_v.at[pl.ds(g * gsub, gsub)]],
                    rows_v.at[pl.ds(g * gsub, gsub)], sem).wait()

            def pt_body(p, _):
                def cb_body(cb, _):
                    o = cb * 16
                    acc = rows_v[p * _K, pl.ds(o, 16)]
                    for j in range(1, _K):
                        acc = jnp.maximum(acc, rows_v[p * _K + j, pl.ds(o, 16)])
                    out_v[p, pl.ds(o, 16)] = acc
                    return 0
                lax.fori_loop(0, c // 16, cb_body, 0)
                return 0
            lax.fori_loop(0, chunk, pt_body, 0)
            pltpu.sync_copy(out_v, out_hbm.at[pl.ds(p0, chunk)])
            return 0
        lax.fori_loop(0, nchunks, chunk_body, 0)

    return k(gidx, xt)


def _degree(dst):
    ec = 128
    wd = 128
    e_per_w = _E // _NW
    nchunks = e_per_w // ec
    rows_per_w = _N // 16
    mesh = plsc.VectorSubcoreMesh(core_axis_name="c", subcore_axis_name="s")

    @functools.partial(
        pl.kernel,
        mesh=mesh,
        out_type=jax.ShapeDtypeStruct((2, _N, wd), jnp.float32),
        scratch_types=[
            pltpu.VMEM((ec,), jnp.int32),
            pltpu.VMEM((ec, wd), jnp.float32),
            pltpu.VMEM_SHARED((_N, wd), jnp.float32),
        ],
    )
    def k(dst_hbm, out_hbm, dst_v, ones_v, table):
        cid = lax.axis_index("c")
        sid = lax.axis_index("s")
        wid = sid * 2 + cid

        def zfill(r, _):
            def zf2(q, _):
                ones_v[r, pl.ds(q * 16, 16)] = jnp.zeros((16,), jnp.float32)
                return 0
            lax.fori_loop(0, wd // 16, zf2, 0)
            return 0
        lax.fori_loop(0, ec, zfill, 0)
        def zcopy(r, _):
            pltpu.sync_copy(ones_v, table.at[pl.ds(sid * rows_per_w + r * ec, ec)])
            return 0
        lax.fori_loop(0, rows_per_w // ec, zcopy, 0)
        def fill_body(r, _):
            def ff2(q, _):
                ones_v[r, pl.ds(q * 16, 16)] = jnp.full((16,), 1.0, jnp.float32)
                return 0
            lax.fori_loop(0, wd // 16, ff2, 0)
            return 0
        lax.fori_loop(0, ec, fill_body, 0)
        plsc.subcore_barrier()

        base = wid * e_per_w
        def chunk_body(ci, _):
            pltpu.sync_copy(dst_hbm.at[pl.ds(base + ci * ec, ec)], dst_v)
            pltpu.sync_copy(ones_v, table.at[dst_v], add=True)
            return 0
        lax.fori_loop(0, nchunks, chunk_body, 0)
        plsc.subcore_barrier()
        pltpu.sync_copy(table.at[pl.ds(sid * rows_per_w, rows_per_w)],
                        out_hbm.at[cid].at[pl.ds(sid * rows_per_w, rows_per_w)])

    return k(dst)


def _edge_agg(h_half, src, dst, width):
    ec = 128
    e_per_w = _E // _NW
    nchunks = e_per_w // ec
    rows_per_w = _N // 16
    zrows = 128
    mesh = plsc.VectorSubcoreMesh(core_axis_name="c", subcore_axis_name="s")

    @functools.partial(
        pl.kernel,
        mesh=mesh,
        out_type=jax.ShapeDtypeStruct((2, _N, width), jnp.float32),
        scratch_types=[
            pltpu.VMEM((ec,), jnp.int32),
            pltpu.VMEM((ec,), jnp.int32),
            pltpu.VMEM((ec, width), jnp.float32),
            pltpu.VMEM((zrows, width), jnp.float32),
            pltpu.VMEM_SHARED((_N, width), jnp.float32),
            pltpu.SemaphoreType.DMA,
        ],
    )
    def k(h_hbm, src_hbm, dst_hbm, out_hbm, src_v, dst_v, rows_v, zbuf, table, sem):
        cid = lax.axis_index("c")
        sid = lax.axis_index("s")
        wid = sid * 2 + cid

        def zfill(r, _):
            def zf2(q, _):
                zbuf[r, pl.ds(q * 16, 16)] = jnp.zeros((16,), jnp.float32)
                return 0
            lax.fori_loop(0, width // 16, zf2, 0)
            return 0
        lax.fori_loop(0, zrows, zfill, 0)
        def zcopy(r, _):
            pltpu.sync_copy(zbuf, table.at[pl.ds(sid * rows_per_w + r * zrows, zrows)])
            return 0
        lax.fori_loop(0, rows_per_w // zrows, zcopy, 0)
        plsc.subcore_barrier()

        base = wid * e_per_w
        def chunk_body(ci, _):
            e0 = base + ci * ec
            pltpu.sync_copy(src_hbm.at[pl.ds(e0, ec)], src_v)
            pltpu.sync_copy(dst_hbm.at[pl.ds(e0, ec)], dst_v)
            pltpu.async_copy(h_hbm.at[src_v], rows_v, sem).wait()
            pltpu.sync_copy(rows_v, table.at[dst_v], add=True)
            return 0
        lax.fori_loop(0, nchunks, chunk_body, 0)
        plsc.subcore_barrier()
        pltpu.sync_copy(table.at[pl.ds(sid * rows_per_w, rows_per_w)],
                        out_hbm.at[cid].at[pl.ds(sid * rows_per_w, rows_per_w)])

    return k(h_half, src, dst)


def _scramble(x):
    c = x.shape[1]
    return jnp.transpose(x.reshape(_B, c, _P), (0, 2, 1)).reshape(_N, c)


def _layer(x, src, dst, deg, w, bias, relu):
    c = x.shape[1]
    y3 = x.reshape(_B, c, _P)
    xt = _scramble(x)
    gidx = _dist_topk(xt, y3).reshape(-1)
    mx = _gather_max(xt, gidx)
    h = _feat_mm(mx, xt, w, deg)
    hw = h.shape[1]
    parts = []
    for j in range(0, hw, 128):
        wd = min(128, hw - j)
        parts.append(_edge_agg(h[:, j:j + wd], src, dst, wd))
    t = jnp.concatenate(parts, axis=2) if len(parts) > 1 else parts[0]
    return _epilogue(t[0], t[1], deg, bias.reshape(1, hw), relu)


def kernel(graph, features, W1, b1, W2, b2, W3, b3):
    src = graph[0]
    dst = graph[1]
    degp = _degree(dst)
    deg = (degp[0, :, 0] + degp[1, :, 0]).reshape(_N, 1)
    h = _layer(features, src, dst, deg, W1, b1, True)
    h = _layer(h, src, dst, deg, W2, b2, True)
    w3p = jnp.pad(W3, ((0, 0), (0, 78)))
    b3p = jnp.pad(b3, (0, 78))
    out = _layer(h, src, dst, deg, w3p, b3p, False)
    return out[:, :50]


# speedup vs baseline: 6.9405x; 1.1477x over previous
import functools

import jax
import jax.numpy as jnp
from jax import lax
from jax.experimental import pallas as pl
from jax.experimental.pallas import tpu as pltpu
from jax.experimental.pallas import tpu_sc as plsc

_B = 4
_P = 2048
_K = 20
_N = _B * _P
_E = 131072
_NW = 32
_NEG = -3.0e38


def _dist_topk_body(rows_ref, full_ref, y_ref, yblk_ref, idx_ref):
    b = pl.program_id(0)
    rows = rows_ref[0]
    full = full_ref[0]
    br = rows.shape[0]
    gram = lax.dot_general(rows, full, (((1,), (1,)), ((), ())),
                           preferred_element_type=jnp.float32)
    y = y_ref[0]
    sq_f = jnp.sum(y * y, axis=0, keepdims=True)
    yb = yblk_ref[0]
    sq_r = jnp.transpose(jnp.sum(yb * yb, axis=0, keepdims=True), (1, 0))
    pd = 2.0 * gram - sq_r - sq_f
    cols = lax.broadcasted_iota(jnp.int32, (br, _P), 1)
    picks = []
    for _ in range(_K):
        m = jnp.max(pd, axis=1, keepdims=True)
        cand = jnp.where(pd >= m, cols, _P)
        a = jnp.min(cand, axis=1, keepdims=True)
        picks.append(a)
        pd = jnp.where(cols == a, _NEG, pd)
    idx_ref[0] = jnp.concatenate(picks, axis=1) + b * _P


def _dist_topk(xt, y3):
    c = xt.shape[1]
    br = 256
    x3 = xt.reshape(_B, _P, c)
    out = pl.pallas_call(
        _dist_topk_body,
        grid=(_B, _P // br),
        in_specs=[
            pl.BlockSpec((1, br, c), lambda b, r: (b, r, 0)),
            pl.BlockSpec((1, _P, c), lambda b, r: (b, 0, 0)),
            pl.BlockSpec((1, c, _P), lambda b, r: (b, 0, 0)),
            pl.BlockSpec((1, c, br), lambda b, r: (b, 0, r)),
        ],
        out_specs=pl.BlockSpec((1, br, _K), lambda b, r: (b, r, 0)),
        out_shape=jax.ShapeDtypeStruct((_B, _P, _K), jnp.int32),
    )(x3, x3, y3, y3)
    return out.reshape(_N, _K)


def _feat_mm_body(mx_ref, xt_ref, w_ref, deg_ref, out_ref):
    c = xt_ref.shape[1]
    d = mx_ref[...] - xt_ref[...]
    h = lax.dot_general(d, w_ref[:c], (((1,), (0,)), ((), ())),
                        preferred_element_type=jnp.float32)
    h += lax.dot_general(xt_ref[...], w_ref[c:], (((1,), (0,)), ((), ())),
                         preferred_element_type=jnp.float32)
    norm = lax.rsqrt(jnp.maximum(deg_ref[...], 1.0))
    out_ref[...] = h * norm


def _feat_mm(mx, xt, w, deg):
    c = xt.shape[1]
    h = w.shape[1]
    bn = 512
    return pl.pallas_call(
        _feat_mm_body,
        grid=(_N // bn,),
        in_specs=[
            pl.BlockSpec((bn, c), lambda i: (i, 0)),
            pl.BlockSpec((bn, c), lambda i: (i, 0)),
            pl.BlockSpec((2 * c, h), lambda i: (0, 0)),
            pl.BlockSpec((bn, 1), lambda i: (i, 0)),
        ],
        out_specs=pl.BlockSpec((bn, h), lambda i: (i, 0)),
        out_shape=jax.ShapeDtypeStruct((_N, h), jnp.float32),
    )(mx, xt, w, deg)


def _epi_body(t0_ref, t1_ref, deg_ref, b_ref, out_ref, *, relu):
    norm = lax.rsqrt(jnp.maximum(deg_ref[...], 1.0))
    o = (t0_ref[...] + t1_ref[...]) * norm + b_ref[...]
    if relu:
        o = jnp.maximum(o, 0.0)
    out_ref[...] = o


def _epilogue(t0, t1, deg, bias, relu):
    h = t0.shape[1]
    bn = 512
    return pl.pallas_call(
        functools.partial(_epi_body, relu=relu),
        grid=(_N // bn,),
        in_specs=[
            pl.BlockSpec((bn, h), lambda i: (i, 0)),
            pl.BlockSpec((bn, h), lambda i: (i, 0)),
            pl.BlockSpec((bn, 1), lambda i: (i, 0)),
            pl.BlockSpec((1, h), lambda i: (0, 0)),
        ],
        out_specs=pl.BlockSpec((bn, h), lambda i: (i, 0)),
        out_shape=jax.ShapeDtypeStruct((_N, h), jnp.float32),
    )(t0, t1, deg, bias)


def _gather_max(xt, gidx):
    c = xt.shape[1]
    chunk = 8 if c > 128 else 16
    gsub = 80
    rows_per_chunk = chunk * _K
    ng = rows_per_chunk // gsub
    pts_per_w = _N // _NW
    nchunks = pts_per_w // chunk
    mesh = plsc.VectorSubcoreMesh(core_axis_name="c", subcore_axis_name="s")

    @functools.partial(
        pl.kernel,
        mesh=mesh,
        out_type=jax.ShapeDtypeStruct((_N, c), jnp.float32),
        scratch_types=[
            pltpu.VMEM((rows_per_chunk,), jnp.int32),
            pltpu.VMEM((rows_per_chunk,), jnp.int32),
            pltpu.VMEM((rows_per_chunk, c), jnp.float32),
            pltpu.VMEM((rows_per_chunk, c), jnp.float32),
            pltpu.VMEM((chunk, c), jnp.float32),
            pltpu.SemaphoreType.DMA,
            pltpu.SemaphoreType.DMA,
        ],
    )
    def k(idx_hbm, xt_hbm, out_hbm, idx_a, idx_b, rows_a, rows_b, out_v,
          sem_a, sem_b):
        wid = lax.axis_index("s") * 2 + lax.axis_index("c")
        base = wid * pts_per_w
        bufs = ((idx_a, rows_a, sem_a), (idx_b, rows_b, sem_b))

        def fire(ci, buf):
            idx_v, rows_v, sem = buf
            p0 = base + ci * chunk
            pltpu.sync_copy(idx_hbm.at[pl.ds(p0 * _K, rows_per_chunk)], idx_v)
            for g in range(ng):
                pltpu.async_copy(
                    xt_hbm.at[idx_v.at[pl.ds(g * gsub, gsub)]],
                    rows_v.at[pl.ds(g * gsub, gsub)], sem)

        def drain(buf):
            idx_v, rows_v, sem = buf
            for g in range(ng):
                pltpu.make_async_copy(
                    xt_hbm.at[idx_v.at[pl.ds(g * gsub, gsub)]],
                    rows_v.at[pl.ds(g * gsub, gsub)], sem).wait()

        fire(0, bufs[0])

        def pair_body(i2, _):
            for bsel in range(2):
                ci = i2 * 2 + bsel
                cur = bufs[bsel]
                nxt = bufs[1 - bsel]

                @pl.when(ci + 1 < nchunks)
                def _():
                    fire(ci + 1, nxt)

                drain(cur)
                rows_v = cur[1]

                def pt_body(p, _):
                    def cb_body(cb, _):
                        o = cb * 16
                        acc = rows_v[p * _K, pl.ds(o, 16)]
                        for j in range(1, _K):
                            acc = jnp.maximum(acc, rows_v[p * _K + j, pl.ds(o, 16)])
                        out_v[p, pl.ds(o, 16)] = acc
                        return 0
                    lax.fori_loop(0, c // 16, cb_body, 0)
                    return 0
                lax.fori_loop(0, chunk, pt_body, 0)
                pltpu.sync_copy(out_v, out_hbm.at[pl.ds(base + ci * chunk, chunk)])
            return 0
        lax.fori_loop(0, nchunks // 2, pair_body, 0)

    return k(gidx, xt)


def _degree(dst):
    ec = 128
    wd = 128
    e_per_w = _E // _NW
    nchunks = e_per_w // ec
    rows_per_w = _N // 16
    mesh = plsc.VectorSubcoreMesh(core_axis_name="c", subcore_axis_name="s")

    @functools.partial(
        pl.kernel,
        mesh=mesh,
        out_type=jax.ShapeDtypeStruct((2, _N, wd), jnp.float32),
        scratch_types=[
            pltpu.VMEM((ec,), jnp.int32),
            pltpu.VMEM((ec, wd), jnp.float32),
            pltpu.VMEM_SHARED((_N, wd), jnp.float32),
        ],
    )
    def k(dst_hbm, out_hbm, dst_v, ones_v, table):
        cid = lax.axis_index("c")
        sid = lax.axis_index("s")
        wid = sid * 2 + cid

        def zfill(r, _):
            def zf2(q, _):
                ones_v[r, pl.ds(q * 16, 16)] = jnp.zeros((16,), jnp.float32)
                return 0
            lax.fori_loop(0, wd // 16, zf2, 0)
            return 0
        lax.fori_loop(0, ec, zfill, 0)
        def zcopy(r, _):
            pltpu.sync_copy(ones_v, table.at[pl.ds(sid * rows_per_w + r * ec, ec)])
            return 0
        lax.fori_loop(0, rows_per_w // ec, zcopy, 0)
        def fill_body(r, _):
            def ff2(q, _):
                ones_v[r, pl.ds(q * 16, 16)] = jnp.full((16,), 1.0, jnp.float32)
                return 0
            lax.fori_loop(0, wd // 16, ff2, 0)
            return 0
        lax.fori_loop(0, ec, fill_body, 0)
        plsc.subcore_barrier()

        base = wid * e_per_w
        def chunk_body(ci, _):
            pltpu.sync_copy(dst_hbm.at[pl.ds(base + ci * ec, ec)], dst_v)
            pltpu.sync_copy(ones_v, table.at[dst_v], add=True)
            return 0
        lax.fori_loop(0, nchunks, chunk_body, 0)
        plsc.subcore_barrier()
        pltpu.sync_copy(table.at[pl.ds(sid * rows_per_w, rows_per_w)],
                        out_hbm.at[cid].at[pl.ds(sid * rows_per_w, rows_per_w)])

    return k(dst)


def _edge_agg(h_half, src, dst, width):
    ec = 128
    e_per_w = _E // _NW
    nchunks = e_per_w // ec
    rows_per_w = _N // 16
    zrows = 128
    mesh = plsc.VectorSubcoreMesh(core_axis_name="c", subcore_axis_name="s")

    @functools.partial(
        pl.kernel,
        mesh=mesh,
        out_type=jax.ShapeDtypeStruct((2, _N, width), jnp.float32),
        scratch_types=[
            pltpu.VMEM((ec,), jnp.int32),
            pltpu.VMEM((ec,), jnp.int32),
            pltpu.VMEM((ec,), jnp.int32),
            pltpu.VMEM((ec,), jnp.int32),
            pltpu.VMEM((ec, width), jnp.float32),
            pltpu.VMEM((ec, width), jnp.float32),
            pltpu.VMEM((zrows, width), jnp.float32),
            pltpu.VMEM_SHARED((_N, width), jnp.float32),
            pltpu.SemaphoreType.DMA,
            pltpu.SemaphoreType.DMA,
        ],
    )
    def k(h_hbm, src_hbm, dst_hbm, out_hbm, src_a, dst_a, src_b, dst_b,
          rows_a, rows_b, zbuf, table, sem_a, sem_b):
        cid = lax.axis_index("c")
        sid = lax.axis_index("s")
        wid = sid * 2 + cid

        def zfill(r, _):
            def zf2(q, _):
                zbuf[r, pl.ds(q * 16, 16)] = jnp.zeros((16,), jnp.float32)
                return 0
            lax.fori_loop(0, width // 16, zf2, 0)
            return 0
        lax.fori_loop(0, zrows, zfill, 0)
        def zcopy(r, _):
            pltpu.sync_copy(zbuf, table.at[pl.ds(sid * rows_per_w + r * zrows, zrows)])
            return 0
        lax.fori_loop(0, rows_per_w // zrows, zcopy, 0)
        plsc.subcore_barrier()

        base = wid * e_per_w
        bufs = ((src_a, dst_a, rows_a, sem_a), (src_b, dst_b, rows_b, sem_b))

        def fire(ci, buf):
            src_v, dst_v, rows_v, sem = buf
            e0 = base + ci * ec
            pltpu.sync_copy(src_hbm.at[pl.ds(e0, ec)], src_v)
            pltpu.sync_copy(dst_hbm.at[pl.ds(e0, ec)], dst_v)
            pltpu.async_copy(h_hbm.at[src_v], rows_v, sem)

        fire(0, bufs[0])

        def pair_body(i2, _):
            for bsel in range(2):
                ci = i2 * 2 + bsel
                cur = bufs[bsel]
                nxt = bufs[1 - bsel]

                @pl.when(ci + 1 < nchunks)
                def _():
                    fire(ci + 1, nxt)

                src_v, dst_v, rows_v, sem = cur
                pltpu.make_async_copy(h_hbm.at[src_v], rows_v, sem).wait()
                pltpu.sync_copy(rows_v, table.at[dst_v], add=True)
            return 0
        lax.fori_loop(0, nchunks // 2, pair_body, 0)
        plsc.subcore_barrier()
        pltpu.sync_copy(table.at[pl.ds(sid * rows_per_w, rows_per_w)],
                        out_hbm.at[cid].at[pl.ds(sid * rows_per_w, rows_per_w)])

    return k(h_half, src, dst)


def _scramble(x):
    c = x.shape[1]
    return jnp.transpose(x.reshape(_B, c, _P), (0, 2, 1)).reshape(_N, c)


def _layer(x, src, dst, deg, w, bias, relu):
    c = x.shape[1]
    y3 = x.reshape(_B, c, _P)
    xt = _scramble(x)
    gidx = _dist_topk(xt, y3).reshape(-1)
    mx = _gather_max(xt, gidx)
    h = _feat_mm(mx, xt, w, deg)
    hw = h.shape[1]
    parts = []
    for j in range(0, hw, 128):
        wd = min(128, hw - j)
        parts.append(_edge_agg(h[:, j:j + wd], src, dst, wd))
    t = jnp.concatenate(parts, axis=2) if len(parts) > 1 else parts[0]
    return _epilogue(t[0], t[1], deg, bias.reshape(1, hw), relu)


def kernel(graph, features, W1, b1, W2, b2, W3, b3):
    src = graph[0]
    dst = graph[1]
    degp = _degree(dst)
    deg = (degp[0, :, 0] + degp[1, :, 0]).reshape(_N, 1)
    h = _layer(features, src, dst, deg, W1, b1, True)
    h = _layer(h, src, dst, deg, W2, b2, True)
    w3p = jnp.pad(W3, ((0, 0), (0, 78)))
    b3p = jnp.pad(b3, (0, 78))
    out = _layer(h, src, dst, deg, w3p, b3p, False)
    return out[:, :50]


# speedup vs baseline: 7.1391x; 1.0286x over previous
import functools

import jax
import jax.numpy as jnp
from jax import lax
from jax.experimental import pallas as pl
from jax.experimental.pallas import tpu as pltpu
from jax.experimental.pallas import tpu_sc as plsc

_B = 4
_P = 2048
_K = 20
_N = _B * _P
_E = 131072
_NW = 32
_NEG = -3.0e38


def _dist_topk_body(rows_ref, full_ref, y_ref, yblk_ref, idx_ref):
    b = pl.program_id(0)
    rows = rows_ref[0]
    full = full_ref[0]
    br = rows.shape[0]
    gram = lax.dot_general(rows, full, (((1,), (1,)), ((), ())),
                           preferred_element_type=jnp.float32)
    y = y_ref[0]
    sq_f = jnp.sum(y * y, axis=0, keepdims=True)
    yb = yblk_ref[0]
    sq_r = jnp.transpose(jnp.sum(yb * yb, axis=0, keepdims=True), (1, 0))
    pd = 2.0 * gram - sq_r - sq_f
    cols = lax.broadcasted_iota(jnp.int32, (br, _P), 1)
    picks = []
    for _ in range(_K):
        m = jnp.max(pd, axis=1, keepdims=True)
        cand = jnp.where(pd >= m, cols, _P)
        a = jnp.min(cand, axis=1, keepdims=True)
        picks.append(a)
        pd = jnp.where(cols == a, _NEG, pd)
    idx_ref[0] = jnp.concatenate(picks, axis=1) + b * _P


def _dist_topk(xt, y3):
    c = xt.shape[1]
    br = 256
    x3 = xt.reshape(_B, _P, c)
    out = pl.pallas_call(
        _dist_topk_body,
        grid=(_B, _P // br),
        in_specs=[
            pl.BlockSpec((1, br, c), lambda b, r: (b, r, 0)),
            pl.BlockSpec((1, _P, c), lambda b, r: (b, 0, 0)),
            pl.BlockSpec((1, c, _P), lambda b, r: (b, 0, 0)),
            pl.BlockSpec((1, c, br), lambda b, r: (b, 0, r)),
        ],
        out_specs=pl.BlockSpec((1, br, _K), lambda b, r: (b, r, 0)),
        out_shape=jax.ShapeDtypeStruct((_B, _P, _K), jnp.int32),
    )(x3, x3, y3, y3)
    return out.reshape(_N, _K)


def _feat_mm_body(mx_ref, xt_ref, w_ref, deg_ref, out_ref):
    c = xt_ref.shape[1]
    d = mx_ref[...] - xt_ref[...]
    h = lax.dot_general(d, w_ref[:c], (((1,), (0,)), ((), ())),
                        preferred_element_type=jnp.float32)
    h += lax.dot_general(xt_ref[...], w_ref[c:], (((1,), (0,)), ((), ())),
                         preferred_element_type=jnp.float32)
    norm = lax.rsqrt(jnp.maximum(deg_ref[...], 1.0))
    out_ref[0] = h * norm


def _feat_mm(mx, xt, w, deg):
    c = xt.shape[1]
    hw = w.shape[1]
    nh = hw // 128
    bn = 512
    return pl.pallas_call(
        _feat_mm_body,
        grid=(nh, _N // bn),
        in_specs=[
            pl.BlockSpec((bn, c), lambda j, i: (i, 0)),
            pl.BlockSpec((bn, c), lambda j, i: (i, 0)),
            pl.BlockSpec((2 * c, 128), lambda j, i: (0, j)),
            pl.BlockSpec((bn, 1), lambda j, i: (i, 0)),
        ],
        out_specs=pl.BlockSpec((1, bn, 128), lambda j, i: (j, i, 0)),
        out_shape=jax.ShapeDtypeStruct((nh, _N, 128), jnp.float32),
    )(mx, xt, w, deg)


def _epi_body(t0_ref, t1_ref, deg_ref, b_ref, out_ref, *, relu):
    norm = lax.rsqrt(jnp.maximum(deg_ref[...], 1.0))
    o = (t0_ref[0] + t1_ref[0]) * norm + b_ref[0]
    if relu:
        o = jnp.maximum(o, 0.0)
    out_ref[...] = o


def _epilogue(t, deg, bias, relu):
    nh = t.shape[0] // 2
    bn = 512
    return pl.pallas_call(
        functools.partial(_epi_body, relu=relu),
        grid=(nh, _N // bn),
        in_specs=[
            pl.BlockSpec((1, bn, 128), lambda j, i: (j, i, 0)),
            pl.BlockSpec((1, bn, 128), lambda j, i: (nh + j, i, 0)),
            pl.BlockSpec((bn, 1), lambda j, i: (i, 0)),
            pl.BlockSpec((1, 1, 128), lambda j, i: (j, 0, 0)),
        ],
        out_specs=pl.BlockSpec((bn, 128), lambda j, i: (i, j)),
        out_shape=jax.ShapeDtypeStruct((_N, nh * 128), jnp.float32),
    )(t, t, deg, bias)


def _gather_max(xt, gidx):
    c = xt.shape[1]
    chunk = 8 if c > 128 else 16
    gsub = 80
    rows_per_chunk = chunk * _K
    ng = rows_per_chunk // gsub
    pts_per_w = _N // _NW
    nchunks = pts_per_w // chunk
    mesh = plsc.VectorSubcoreMesh(core_axis_name="c", subcore_axis_name="s")

    @functools.partial(
        pl.kernel,
        mesh=mesh,
        out_type=jax.ShapeDtypeStruct((_N, c), jnp.float32),
        scratch_types=[
            pltpu.VMEM((rows_per_chunk,), jnp.int32),
            pltpu.VMEM((rows_per_chunk,), jnp.int32),
            pltpu.VMEM((rows_per_chunk, c), jnp.float32),
            pltpu.VMEM((rows_per_chunk, c), jnp.float32),
            pltpu.VMEM((chunk, c), jnp.float32),
            pltpu.SemaphoreType.DMA,
            pltpu.SemaphoreType.DMA,
        ],
    )
    def k(idx_hbm, xt_hbm, out_hbm, idx_a, idx_b, rows_a, rows_b, out_v,
          sem_a, sem_b):
        wid = lax.axis_index("s") * 2 + lax.axis_index("c")
        base = wid * pts_per_w
        bufs = ((idx_a, rows_a, sem_a), (idx_b, rows_b, sem_b))

        def fire(ci, buf):
            idx_v, rows_v, sem = buf
            p0 = base + ci * chunk
            pltpu.sync_copy(idx_hbm.at[pl.ds(p0 * _K, rows_per_chunk)], idx_v)
            for g in range(ng):
                pltpu.async_copy(
                    xt_hbm.at[idx_v.at[pl.ds(g * gsub, gsub)]],
                    rows_v.at[pl.ds(g * gsub, gsub)], sem)

        def drain(buf):
            idx_v, rows_v, sem = buf
            for g in range(ng):
                pltpu.make_async_copy(
                    xt_hbm.at[idx_v.at[pl.ds(g * gsub, gsub)]],
                    rows_v.at[pl.ds(g * gsub, gsub)], sem).wait()

        fire(0, bufs[0])

        def pair_body(i2, _):
            for bsel in range(2):
                ci = i2 * 2 + bsel
                cur = bufs[bsel]
                nxt = bufs[1 - bsel]

                @pl.when(ci + 1 < nchunks)
                def _():
                    fire(ci + 1, nxt)

                drain(cur)
                rows_v = cur[1]

                def pt_body(p, _):
                    for cb in range(c // 16):
                        o = cb * 16
                        acc = rows_v[p * _K, pl.ds(o, 16)]
                        for j in range(1, _K):
                            acc = jnp.maximum(acc, rows_v[p * _K + j, pl.ds(o, 16)])
                        out_v[p, pl.ds(o, 16)] = acc
                    return 0
                lax.fori_loop(0, chunk, pt_body, 0)
                pltpu.sync_copy(out_v, out_hbm.at[pl.ds(base + ci * chunk, chunk)])
            return 0
        lax.fori_loop(0, nchunks // 2, pair_body, 0)

    return k(gidx, xt)


def _degree(dst):
    ec = 128
    wd = 128
    e_per_w = _E // _NW
    nchunks = e_per_w // ec
    rows_per_w = _N // 16
    mesh = plsc.VectorSubcoreMesh(core_axis_name="c", subcore_axis_name="s")

    @functools.partial(
        pl.kernel,
        mesh=mesh,
        out_type=jax.ShapeDtypeStruct((2, _N, wd), jnp.float32),
        scratch_types=[
            pltpu.VMEM((ec,), jnp.int32),
            pltpu.VMEM((ec, wd), jnp.float32),
            pltpu.VMEM_SHARED((_N, wd), jnp.float32),
        ],
    )
    def k(dst_hbm, out_hbm, dst_v, ones_v, table):
        cid = lax.axis_index("c")
        sid = lax.axis_index("s")
        wid = sid * 2 + cid

        def zfill(r, _):
            def zf2(q, _):
                ones_v[r, pl.ds(q * 16, 16)] = jnp.zeros((16,), jnp.float32)
                return 0
            lax.fori_loop(0, wd // 16, zf2, 0)
            return 0
        lax.fori_loop(0, ec, zfill, 0)
        def zcopy(r, _):
            pltpu.sync_copy(ones_v, table.at[pl.ds(sid * rows_per_w + r * ec, ec)])
            return 0
        lax.fori_loop(0, rows_per_w // ec, zcopy, 0)
        def fill_body(r, _):
            def ff2(q, _):
                ones_v[r, pl.ds(q * 16, 16)] = jnp.full((16,), 1.0, jnp.float32)
                return 0
            lax.fori_loop(0, wd // 16, ff2, 0)
            return 0
        lax.fori_loop(0, ec, fill_body, 0)
        plsc.subcore_barrier()

        base = wid * e_per_w
        def chunk_body(ci, _):
            pltpu.sync_copy(dst_hbm.at[pl.ds(base + ci * ec, ec)], dst_v)
            pltpu.sync_copy(ones_v, table.at[dst_v], add=True)
            return 0
        lax.fori_loop(0, nchunks, chunk_body, 0)
        plsc.subcore_barrier()
        pltpu.sync_copy(table.at[pl.ds(sid * rows_per_w, rows_per_w)],
                        out_hbm.at[cid].at[pl.ds(sid * rows_per_w, rows_per_w)])

    return k(dst)


def _edge_agg(h_stacked, src, dst):
    nh = h_stacked.shape[0]
    width = 128
    ec = 128
    e_per_w = _E // _NW
    nchunks = e_per_w // ec
    rows_per_w = _N // 16
    zrows = 128
    mesh = plsc.VectorSubcoreMesh(core_axis_name="c", subcore_axis_name="s")

    @functools.partial(
        pl.kernel,
        mesh=mesh,
        out_type=jax.ShapeDtypeStruct((2 * nh, _N, width), jnp.float32),
        scratch_types=[
            pltpu.VMEM((ec,), jnp.int32),
            pltpu.VMEM((ec,), jnp.int32),
            pltpu.VMEM((ec,), jnp.int32),
            pltpu.VMEM((ec,), jnp.int32),
            pltpu.VMEM((ec, width), jnp.float32),
            pltpu.VMEM((ec, width), jnp.float32),
            pltpu.VMEM((zrows, width), jnp.float32),
            pltpu.VMEM_SHARED((_N, width), jnp.float32),
            pltpu.SemaphoreType.DMA,
            pltpu.SemaphoreType.DMA,
        ],
    )
    def k(h_hbm, src_hbm, dst_hbm, out_hbm, src_a, dst_a, src_b, dst_b,
          rows_a, rows_b, zbuf, table, sem_a, sem_b):
        cid = lax.axis_index("c")
        sid = lax.axis_index("s")
        wid = sid * 2 + cid

        def zfill(r, _):
            def zf2(q, _):
                zbuf[r, pl.ds(q * 16, 16)] = jnp.zeros((16,), jnp.float32)
                return 0
            lax.fori_loop(0, width // 16, zf2, 0)
            return 0
        lax.fori_loop(0, zrows, zfill, 0)

        def zero_table():
            def zcopy(r, _):
                pltpu.sync_copy(zbuf, table.at[pl.ds(sid * rows_per_w + r * zrows, zrows)])
                return 0
            lax.fori_loop(0, rows_per_w // zrows, zcopy, 0)

        base = wid * e_per_w
        bufs = ((src_a, dst_a, rows_a, sem_a), (src_b, dst_b, rows_b, sem_b))

        for jh in range(nh):
            h_j = h_hbm.at[jh]
            zero_table()
            plsc.subcore_barrier()

            def fire(ci, buf):
                src_v, dst_v, rows_v, sem = buf
                e0 = base + ci * ec
                pltpu.sync_copy(src_hbm.at[pl.ds(e0, ec)], src_v)
                pltpu.sync_copy(dst_hbm.at[pl.ds(e0, ec)], dst_v)
                pltpu.async_copy(h_j.at[src_v], rows_v, sem)

            fire(0, bufs[0])

            def pair_body(i2, _):
                for bsel in range(2):
                    ci = i2 * 2 + bsel
                    cur = bufs[bsel]
                    nxt = bufs[1 - bsel]

                    @pl.when(ci + 1 < nchunks)
                    def _():
                        fire(ci + 1, nxt)

                    src_v, dst_v, rows_v, sem = cur
                    pltpu.make_async_copy(h_j.at[src_v], rows_v, sem).wait()
                    pltpu.sync_copy(rows_v, table.at[dst_v], add=True)
                return 0
            lax.fori_loop(0, nchunks // 2, pair_body, 0)
            plsc.subcore_barrier()
            pltpu.sync_copy(table.at[pl.ds(sid * rows_per_w, rows_per_w)],
                            out_hbm.at[cid * nh + jh].at[pl.ds(sid * rows_per_w, rows_per_w)])

    return k(h_stacked, src, dst)


def _scramble(x):
    c = x.shape[1]
    return jnp.transpose(x.reshape(_B, c, _P), (0, 2, 1)).reshape(_N, c)


def _layer(x, src, dst, deg, w, bias, relu):
    c = x.shape[1]
    y3 = x.reshape(_B, c, _P)
    xt = _scramble(x)
    gidx = _dist_topk(xt, y3).reshape(-1)
    mx = _gather_max(xt, gidx)
    h = _feat_mm(mx, xt, w, deg)
    nh = h.shape[0]
    t = _edge_agg(h, src, dst)
    return _epilogue(t, deg, bias.reshape(nh, 1, 128), relu)


def kernel(graph, features, W1, b1, W2, b2, W3, b3):
    src = graph[0]
    dst = graph[1]
    degp = _degree(dst)
    deg = (degp[0, :, 0] + degp[1, :, 0]).reshape(_N, 1)
    h = _layer(features, src, dst, deg, W1, b1, True)
    h = _layer(h, src, dst, deg, W2, b2, True)
    w3p = jnp.pad(W3, ((0, 0), (0, 78)))
    b3p = jnp.pad(b3, (0, 78))
    out = _layer(h, src, dst, deg, w3p, b3p, False)
    return out[:, :50]


# speedup vs baseline: 8.1309x; 1.1389x over previous
import functools

import jax
import jax.numpy as jnp
from jax import lax
from jax.experimental import pallas as pl
from jax.experimental.pallas import tpu as pltpu
from jax.experimental.pallas import tpu_sc as plsc

_B = 4
_P = 2048
_K = 20
_N = _B * _P
_E = 131072
_NW = 32
_NEG = -3.0e38


def _dist_topk_body(rows_ref, full_ref, y_ref, yblk_ref, idx_ref):
    b = pl.program_id(0)
    rows = rows_ref[0]
    full = full_ref[0]
    br = rows.shape[0]
    gram = lax.dot_general(rows, full, (((1,), (1,)), ((), ())),
                           preferred_element_type=jnp.float32)
    y = y_ref[0]
    sq_f = jnp.sum(y * y, axis=0, keepdims=True)
    yb = yblk_ref[0]
    sq_r = jnp.transpose(jnp.sum(yb * yb, axis=0, keepdims=True), (1, 0))
    pd = 2.0 * gram - sq_r - sq_f
    cols = lax.broadcasted_iota(jnp.int32, (br, _P), 1)
    picks = []
    for _ in range(_K):
        a = jnp.argmax(pd, axis=1).astype(jnp.int32)[:, None]
        picks.append(a)
        pd = jnp.where(cols == a, _NEG, pd)
    idx_ref[0] = jnp.concatenate(picks, axis=1) + b * _P


def _dist_topk(xt, y3):
    c = xt.shape[1]
    br = 256
    x3 = xt.reshape(_B, _P, c)
    out = pl.pallas_call(
        _dist_topk_body,
        grid=(_B, _P // br),
        in_specs=[
            pl.BlockSpec((1, br, c), lambda b, r: (b, r, 0)),
            pl.BlockSpec((1, _P, c), lambda b, r: (b, 0, 0)),
            pl.BlockSpec((1, c, _P), lambda b, r: (b, 0, 0)),
            pl.BlockSpec((1, c, br), lambda b, r: (b, 0, r)),
        ],
        out_specs=pl.BlockSpec((1, br, _K), lambda b, r: (b, r, 0)),
        out_shape=jax.ShapeDtypeStruct((_B, _P, _K), jnp.int32),
    )(x3, x3, y3, y3)
    return out.reshape(_N, _K)


def _feat_mm_body(mx_ref, xt_ref, w_ref, deg_ref, out_ref):
    c = xt_ref.shape[1]
    d = mx_ref[...] - xt_ref[...]
    h = lax.dot_general(d, w_ref[:c], (((1,), (0,)), ((), ())),
                        preferred_element_type=jnp.float32)
    h += lax.dot_general(xt_ref[...], w_ref[c:], (((1,), (0,)), ((), ())),
                         preferred_element_type=jnp.float32)
    norm = lax.rsqrt(jnp.maximum(deg_ref[...], 1.0))
    out_ref[0] = h * norm


def _feat_mm(mx, xt, w, deg):
    c = xt.shape[1]
    hw = w.shape[1]
    nh = hw // 128
    bn = 512
    return pl.pallas_call(
        _feat_mm_body,
        grid=(nh, _N // bn),
        in_specs=[
            pl.BlockSpec((bn, c), lambda j, i: (i, 0)),
            pl.BlockSpec((bn, c), lambda j, i: (i, 0)),
            pl.BlockSpec((2 * c, 128), lambda j, i: (0, j)),
            pl.BlockSpec((bn, 1), lambda j, i: (i, 0)),
        ],
        out_specs=pl.BlockSpec((1, bn, 128), lambda j, i: (j, i, 0)),
        out_shape=jax.ShapeDtypeStruct((nh, _N, 128), jnp.float32),
    )(mx, xt, w, deg)


def _epi_body(t0_ref, t1_ref, deg_ref, b_ref, out_ref, *, relu):
    norm = lax.rsqrt(jnp.maximum(deg_ref[...], 1.0))
    o = (t0_ref[0] + t1_ref[0]) * norm + b_ref[0]
    if relu:
        o = jnp.maximum(o, 0.0)
    out_ref[...] = o


def _epilogue(t, deg, bias, relu):
    nh = t.shape[0] // 2
    bn = 512
    return pl.pallas_call(
        functools.partial(_epi_body, relu=relu),
        grid=(nh, _N // bn),
        in_specs=[
            pl.BlockSpec((1, bn, 128), lambda j, i: (j, i, 0)),
            pl.BlockSpec((1, bn, 128), lambda j, i: (nh + j, i, 0)),
            pl.BlockSpec((bn, 1), lambda j, i: (i, 0)),
            pl.BlockSpec((1, 1, 128), lambda j, i: (j, 0, 0)),
        ],
        out_specs=pl.BlockSpec((bn, 128), lambda j, i: (i, j)),
        out_shape=jax.ShapeDtypeStruct((_N, nh * 128), jnp.float32),
    )(t, t, deg, bias)


def _gather_max(xt, gidx):
    c = xt.shape[1]
    chunk = 8 if c > 128 else 16
    gsub = 80
    rows_per_chunk = chunk * _K
    ng = rows_per_chunk // gsub
    pts_per_w = _N // _NW
    nchunks = pts_per_w // chunk
    mesh = plsc.VectorSubcoreMesh(core_axis_name="c", subcore_axis_name="s")

    @functools.partial(
        pl.kernel,
        mesh=mesh,
        out_type=jax.ShapeDtypeStruct((_N, c), jnp.float32),
    scratch_types=[
            pltpu.VMEM((pts_per_w * _K,), jnp.int32),
            pltpu.VMEM((rows_per_chunk, c), jnp.float32),
            pltpu.VMEM((rows_per_chunk, c), jnp.float32),
            pltpu.VMEM((chunk, c), jnp.float32),
            pltpu.SemaphoreType.DMA,
            pltpu.SemaphoreType.DMA,
        ],
    )
    def k(idx_hbm, xt_hbm, out_hbm, idx_all, rows_a, rows_b, out_v,
          sem_a, sem_b):
        wid = lax.axis_index("s") * 2 + lax.axis_index("c")
        base = wid * pts_per_w
        pltpu.sync_copy(idx_hbm.at[pl.ds(base * _K, pts_per_w * _K)], idx_all)
        bufs = ((rows_a, sem_a), (rows_b, sem_b))

        def fire(ci, buf):
            rows_v, sem = buf
            for g in range(ng):
                pltpu.async_copy(
                    xt_hbm.at[idx_all.at[pl.ds(ci * rows_per_chunk + g * gsub, gsub)]],
                    rows_v.at[pl.ds(g * gsub, gsub)], sem)

        def drain(ci, buf):
            rows_v, sem = buf
            for g in range(ng):
                pltpu.make_async_copy(
                    xt_hbm.at[idx_all.at[pl.ds(ci * rows_per_chunk + g * gsub, gsub)]],
                    rows_v.at[pl.ds(g * gsub, gsub)], sem).wait()

        fire(0, bufs[0])

        def pair_body(i2, _):
            for bsel in range(2):
                ci = i2 * 2 + bsel
                cur = bufs[bsel]
                nxt = bufs[1 - bsel]

                @pl.when(ci + 1 < nchunks)
                def _():
                    fire(ci + 1, nxt)

                drain(ci, cur)
                rows_v = cur[0]

                def pt_body(p, _):
                    for cb in range(c // 16):
                        o = cb * 16
                        acc = rows_v[p * _K, pl.ds(o, 16)]
                        for j in range(1, _K):
                            acc = jnp.maximum(acc, rows_v[p * _K + j, pl.ds(o, 16)])
                        out_v[p, pl.ds(o, 16)] = acc
                    return 0
                lax.fori_loop(0, chunk, pt_body, 0)
                pltpu.sync_copy(out_v, out_hbm.at[pl.ds(base + ci * chunk, chunk)])
            return 0
        lax.fori_loop(0, nchunks // 2, pair_body, 0)

    return k(gidx, xt)


def _degree(dst):
    ec = 128
    wd = 128
    e_per_w = _E // _NW
    nchunks = e_per_w // ec
    rows_per_w = _N // 16
    mesh = plsc.VectorSubcoreMesh(core_axis_name="c", subcore_axis_name="s")

    @functools.partial(
        pl.kernel,
        mesh=mesh,
        out_type=jax.ShapeDtypeStruct((2, _N, wd), jnp.float32),
        scratch_types=[
            pltpu.VMEM((ec,), jnp.int32),
            pltpu.VMEM((ec, wd), jnp.float32),
            pltpu.VMEM_SHARED((_N, wd), jnp.float32),
        ],
    )
    def k(dst_hbm, out_hbm, dst_v, ones_v, table):
        cid = lax.axis_index("c")
        sid = lax.axis_index("s")
        wid = sid * 2 + cid

        def zfill(r, _):
            def zf2(q, _):
                ones_v[r, pl.ds(q * 16, 16)] = jnp.zeros((16,), jnp.float32)
                return 0
            lax.fori_loop(0, wd // 16, zf2, 0)
            return 0
        lax.fori_loop(0, ec, zfill, 0)
        def zcopy(r, _):
            pltpu.sync_copy(ones_v, table.at[pl.ds(sid * rows_per_w + r * ec, ec)])
            return 0
        lax.fori_loop(0, rows_per_w // ec, zcopy, 0)
        def fill_body(r, _):
            def ff2(q, _):
                ones_v[r, pl.ds(q * 16, 16)] = jnp.full((16,), 1.0, jnp.float32)
                return 0
            lax.fori_loop(0, wd // 16, ff2, 0)
            return 0
        lax.fori_loop(0, ec, fill_body, 0)
        plsc.subcore_barrier()

        base = wid * e_per_w
        def chunk_body(ci, _):
            pltpu.sync_copy(dst_hbm.at[pl.ds(base + ci * ec, ec)], dst_v)
            pltpu.sync_copy(ones_v, table.at[dst_v], add=True)
            return 0
        lax.fori_loop(0, nchunks, chunk_body, 0)
        plsc.subcore_barrier()
        pltpu.sync_copy(table.at[pl.ds(sid * rows_per_w, rows_per_w)],
                        out_hbm.at[cid].at[pl.ds(sid * rows_per_w, rows_per_w)])

    return k(dst)


def _edge_agg(h_stacked, src, dst):
    nh = h_stacked.shape[0]
    width = 128
    ec = 128
    e_per_w = _E // _NW
    nchunks = e_per_w // ec
    rows_per_w = _N // 16
    zrows = 128
    mesh = plsc.VectorSubcoreMesh(core_axis_name="c", subcore_axis_name="s")

    @functools.partial(
        pl.kernel,
        mesh=mesh,
        out_type=jax.ShapeDtypeStruct((2 * nh, _N, width), jnp.float32),
        scratch_types=[
            pltpu.VMEM((e_per_w,), jnp.int32),
            pltpu.VMEM((nchunks, ec), jnp.int32),
            pltpu.VMEM((ec, width), jnp.float32),
            pltpu.VMEM((ec, width), jnp.float32),
            pltpu.VMEM((zrows, width), jnp.float32),
            pltpu.VMEM_SHARED((_N, width), jnp.float32),
            pltpu.SemaphoreType.DMA,
            pltpu.SemaphoreType.DMA,
        ],
    )
    def k(h_hbm, src_hbm, dst_hbm, out_hbm, src_all, dst_all,
          rows_a, rows_b, zbuf, table, sem_a, sem_b):
        cid = lax.axis_index("c")
        sid = lax.axis_index("s")
        wid = sid * 2 + cid
        pltpu.sync_copy(src_hbm.at[pl.ds(wid * e_per_w, e_per_w)], src_all)
        pltpu.sync_copy(dst_hbm.at[wid], dst_all)

        def zfill(r, _):
            def zf2(q, _):
                zbuf[r, pl.ds(q * 16, 16)] = jnp.zeros((16,), jnp.float32)
                return 0
            lax.fori_loop(0, width // 16, zf2, 0)
            return 0
        lax.fori_loop(0, zrows, zfill, 0)

        def zero_table():
            def zcopy(r, _):
                pltpu.sync_copy(zbuf, table.at[pl.ds(sid * rows_per_w + r * zrows, zrows)])
                return 0
            lax.fori_loop(0, rows_per_w // zrows, zcopy, 0)

        bufs = ((rows_a, sem_a), (rows_b, sem_b))

        for jh in range(nh):
            h_j = h_hbm.at[jh]
            zero_table()
            plsc.subcore_barrier()

            def fire(ci, buf):
                rows_v, sem = buf
                pltpu.async_copy(h_j.at[src_all.at[pl.ds(ci * ec, ec)]],
                                 rows_v, sem)

            fire(0, bufs[0])

            def pair_body(i2, _):
                for bsel in range(2):
                    ci = i2 * 2 + bsel
                    cur = bufs[bsel]
                    nxt = bufs[1 - bsel]

                    @pl.when(ci + 1 < nchunks)
                    def _():
                        fire(ci + 1, nxt)

                    rows_v, sem = cur
                    pltpu.make_async_copy(
                        h_j.at[src_all.at[pl.ds(ci * ec, ec)]],
                        rows_v, sem).wait()
                    pltpu.sync_copy(rows_v, table.at[dst_all.at[ci]], add=True)
                return 0
            lax.fori_loop(0, nchunks // 2, pair_body, 0)
            plsc.subcore_barrier()
            pltpu.sync_copy(table.at[pl.ds(sid * rows_per_w, rows_per_w)],
                            out_hbm.at[cid * nh + jh].at[pl.ds(sid * rows_per_w, rows_per_w)])

    return k(h_stacked, src, dst.reshape(_NW, nchunks, ec))


def _scramble(x):
    c = x.shape[1]
    return jnp.transpose(x.reshape(_B, c, _P), (0, 2, 1)).reshape(_N, c)


def _layer(x, src, dst, deg, w, bias, relu):
    c = x.shape[1]
    y3 = x.reshape(_B, c, _P)
    xt = _scramble(x)
    gidx = _dist_topk(xt, y3).reshape(-1)
    mx = _gather_max(xt, gidx)
    h = _feat_mm(mx, xt, w, deg)
    nh = h.shape[0]
    t = _edge_agg(h, src, dst)
    return _epilogue(t, deg, bias.reshape(nh, 1, 128), relu)


def kernel(graph, features, W1, b1, W2, b2, W3, b3):
    src = graph[0]
    dst = graph[1]
    degp = _degree(dst)
    deg = (degp[0, :, 0] + degp[1, :, 0]).reshape(_N, 1)
    h = _layer(features, src, dst, deg, W1, b1, True)
    h = _layer(h, src, dst, deg, W2, b2, True)
    w3p = jnp.pad(W3, ((0, 0), (0, 78)))
    b3p = jnp.pad(b3, (0, 78))
    out = _layer(h, src, dst, deg, w3p, b3p, False)
    return out[:, :50]
